# baseline (speedup 1.0000x reference)
import jax
import jax.numpy as jnp
from jax.experimental import pallas as pl
from jax.experimental.pallas import tpu as pltpu

K = 512
D = 256
BETA = 0.25

_TILE = 4608
_TPB = 13824 // _TILE


def _vq_body(x_ref, w_ref, enc_ref, q_ref, loss_ref):
    b = pl.program_id(0)
    t = pl.program_id(1)
    x_cm = x_ref[0]
    ir = jax.lax.broadcasted_iota(jnp.int32, (D, D), 0)
    ic = jax.lax.broadcasted_iota(jnp.int32, (D, D), 1)
    eye = (ir == ic).astype(jnp.float32)
    xt = jax.lax.dot_general(x_cm, eye, (((0,), (0,)), ((), ())),
                             preferred_element_type=jnp.float32)
    w = w_ref[...]
    x_sq = jnp.sum(xt * xt, axis=1, keepdims=True)
    w_sq = jnp.sum(w * w, axis=1)
    mm = jax.lax.dot_general(xt, w, (((1,), (1,)), ((), ())),
                             preferred_element_type=jnp.float32)
    d = x_sq + w_sq - 2.0 * mm
    dmin = jnp.min(d, axis=1, keepdims=True)
    iota_f = jax.lax.broadcasted_iota(jnp.int32, (1, K), 1).astype(jnp.float32)
    idx = jnp.min(jnp.where(d == dmin, iota_f, float(K)),
                  axis=1, keepdims=True)
    enc = (iota_f == idx).astype(jnp.float32)
    enc_ref[...] = enc
    q_ref[0] = jax.lax.dot_general(w, enc, (((0,), (1,)), ((), ())),
                                   preferred_element_type=jnp.float32)

    @pl.when((b == 0) & (t == 0))
    def _():
        loss_ref[...] = jnp.zeros((1, 1), jnp.float32)

    ones_row = jnp.ones((1, _TILE), jnp.float32)
    loss_ref[...] += jax.lax.dot_general(
        ones_row, dmin, (((1,), (0,)), ((), ())),
        preferred_element_type=jnp.float32)


def kernel(x, W):
    B, C, D1, D2, D3 = x.shape
    S = D1 * D2 * D3
    N = B * S
    x3 = x.reshape(B, C, S)
    enc, quant, loss_sum = pl.pallas_call(
        _vq_body,
        grid=(B, _TPB),
        in_specs=[
            pl.BlockSpec((1, D, _TILE), lambda b, t: (b, 0, t)),
            pl.BlockSpec((K, D), lambda b, t: (0, 0)),
        ],
        out_specs=[
            pl.BlockSpec((_TILE, K), lambda b, t: (b * _TPB + t, 0)),
            pl.BlockSpec((1, D, _TILE), lambda b, t: (b, 0, t)),
            pl.BlockSpec((1, 1), lambda b, t: (0, 0)),
        ],
        out_shape=[
            jax.ShapeDtypeStruct((N, K), jnp.float32),
            jax.ShapeDtypeStruct((B, D, S), jnp.float32),
            jax.ShapeDtypeStruct((1, 1), jnp.float32),
        ],
    )(x3, W)
    mse = loss_sum[0, 0] / (N * D)
    e_latent = jnp.clip(mse, 0.0, 10.0)
    loss = e_latent + BETA * e_latent
    out = quant.reshape(B, C, D1, D2, D3)
    return (loss, out, enc)


# speedup vs baseline: 2.2630x; 2.2630x over previous
import jax
import jax.numpy as jnp
from jax.experimental import pallas as pl
from jax.experimental.pallas import tpu as pltpu

K = 512
D = 256
BETA = 0.25

_TILE = 4608
_TPB = 13824 // _TILE


def _vq_body(x_ref, w_ref, enc_ref, q_ref, loss_ref):
    b = pl.program_id(0)
    t = pl.program_id(1)
    xt = x_ref[...]
    w = w_ref[...]
    x_sq = jnp.sum(xt * xt, axis=1, keepdims=True)
    w_sq = jnp.sum(w * w, axis=1)
    mmn = jax.lax.dot_general(xt, w * -2.0, (((1,), (1,)), ((), ())),
                              preferred_element_type=jnp.float32)
    d = x_sq + w_sq + mmn
    dmin = jnp.min(d, axis=1, keepdims=True)
    iota_f = jax.lax.broadcasted_iota(jnp.int32, (1, K), 1).astype(jnp.float32)
    idx = jnp.min(jnp.where(d == dmin, iota_f, float(K)),
                  axis=1, keepdims=True)
    enc = (iota_f == idx).astype(jnp.float32)
    enc_ref[...] = enc
    q_ref[...] = jax.lax.dot_general(enc, w, (((1,), (0,)), ((), ())),
                                     preferred_element_type=jnp.float32)

    @pl.when((b == 0) & (t == 0))
    def _():
        loss_ref[...] = jnp.zeros((1, 1), jnp.float32)

    ones_row = jnp.ones((1, _TILE), jnp.float32)
    loss_ref[...] += jax.lax.dot_general(
        ones_row, dmin, (((1,), (0,)), ((), ())),
        preferred_element_type=jnp.float32)


def kernel(x, W):
    B, C, D1, D2, D3 = x.shape
    S = D1 * D2 * D3
    N = B * S
    x_flat = jnp.transpose(x, (0, 2, 3, 4, 1)).reshape(N, D)
    enc, quant, loss_sum = pl.pallas_call(
        _vq_body,
        grid=(B, _TPB),
        in_specs=[
            pl.BlockSpec((_TILE, D), lambda b, t: (b * _TPB + t, 0)),
            pl.BlockSpec((K, D), lambda b, t: (0, 0)),
        ],
        out_specs=[
            pl.BlockSpec((_TILE, K), lambda b, t: (b * _TPB + t, 0)),
            pl.BlockSpec((_TILE, D), lambda b, t: (b * _TPB + t, 0)),
            pl.BlockSpec((1, 1), lambda b, t: (0, 0)),
        ],
        out_shape=[
            jax.ShapeDtypeStruct((N, K), jnp.float32),
            jax.ShapeDtypeStruct((N, D), jnp.float32),
            jax.ShapeDtypeStruct((1, 1), jnp.float32),
        ],
    )(x_flat, W)
    mse = loss_sum[0, 0] / (N * D)
    e_latent = jnp.clip(mse, 0.0, 10.0)
    loss = e_latent + BETA * e_latent
    out = jnp.transpose(quant.reshape(B, D1, D2, D3, C), (0, 4, 1, 2, 3))
    return (loss, out, enc)


# speedup vs baseline: 2.2658x; 1.0013x over previous
import jax
import jax.numpy as jnp
from jax.experimental import pallas as pl

K = 512
D = 256
BETA = 0.25

_TILE = 4608
_TPB = 13824 // _TILE


def _vq_body(x_ref, w_ref, enc_ref, q_ref, loss_ref):
    b = pl.program_id(0)
    t = pl.program_id(1)
    xt = x_ref[...]
    w = w_ref[...]
    x_sq = jnp.sum(xt * xt, axis=1, keepdims=True)
    w_sq = jnp.sum(w * w, axis=1)
    mmn = jax.lax.dot_general(xt, w * -2.0, (((1,), (1,)), ((), ())),
                              preferred_element_type=jnp.float32)
    d = x_sq + w_sq + mmn
    dmin = jnp.min(d, axis=1, keepdims=True)
    iota_f = jax.lax.broadcasted_iota(jnp.int32, (1, K), 1).astype(jnp.float32)
    idx = jnp.min(jnp.where(d == dmin, iota_f, float(K)),
                  axis=1, keepdims=True)
    enc = (iota_f == idx).astype(jnp.float32)
    enc_ref[...] = enc
    q_ref[...] = jax.lax.dot_general(enc, w, (((1,), (0,)), ((), ())),
                                     preferred_element_type=jnp.float32)

    @pl.when((b == 0) & (t == 0))
    def _():
        loss_ref[...] = jnp.zeros((1, 1), jnp.float32)

    ones_row = jnp.ones((1, _TILE), jnp.float32)
    loss_ref[...] += jax.lax.dot_general(
        ones_row, dmin, (((1,), (0,)), ((), ())),
        preferred_element_type=jnp.float32)


def kernel(x, W):
    B, C, D1, D2, D3 = x.shape
    S = D1 * D2 * D3
    N = B * S
    x_flat = jnp.transpose(x, (0, 2, 3, 4, 1)).reshape(N, D)
    enc, quant, loss_sum = pl.pallas_call(
        _vq_body,
        grid=(B, _TPB),
        in_specs=[
            pl.BlockSpec((_TILE, D), lambda b, t: (b * _TPB + t, 0)),
            pl.BlockSpec((K, D), lambda b, t: (0, 0)),
        ],
        out_specs=[
            pl.BlockSpec((_TILE, K), lambda b, t: (b * _TPB + t, 0)),
            pl.BlockSpec((_TILE, D), lambda b, t: (b * _TPB + t, 0)),
            pl.BlockSpec((1, 1), lambda b, t: (0, 0)),
        ],
        out_shape=[
            jax.ShapeDtypeStruct((N, K), jnp.float32),
            jax.ShapeDtypeStruct((N, D), jnp.float32),
            jax.ShapeDtypeStruct((1, 1), jnp.float32),
        ],
    )(x_flat, W)
    mse = loss_sum[0, 0] / (N * D)
    e_latent = jnp.clip(mse, 0.0, 10.0)
    loss = e_latent + BETA * e_latent
    out = jnp.transpose(quant.reshape(B, D1, D2, D3, C), (0, 4, 1, 2, 3))
    return (loss, out, enc)
